# baseline (speedup 1.0000x reference)
import functools

import jax
import jax.numpy as jnp
from jax import lax
from jax.experimental import pallas as pl
from jax.experimental.pallas import tpu as pltpu
from jax.experimental.pallas import tpu_sc as plsc

N = 10000
E = 320000
D = 128

NC = 2
NS = 16
NW = NC * NS
EPT = E // NW
RCH = 624
DEG_CH = 624


def _make_sc_agg(with_deg: bool, C: int, R: int):
    NCHUNK = EPT // C
    out_type = [jax.ShapeDtypeStruct((N, NC * D), jnp.float32)]
    scratch = (
        [pltpu.VMEM((C,), jnp.int32) for _ in range(R)]
        + [pltpu.VMEM((C,), jnp.int32) for _ in range(R)]
        + [pltpu.VMEM((C, D), jnp.float32) for _ in range(R)]
        + [pltpu.VMEM_SHARED((N, D), jnp.float32)]
        + [pltpu.SemaphoreType.DMA for _ in range(3 * R)]
    )
    if with_deg:
        out_type.append(jax.ShapeDtypeStruct((N,), jnp.float32))
        out_type.append(jax.ShapeDtypeStruct((N,), jnp.float32))
        scratch += [
            pltpu.VMEM((C,), jnp.float32),
            pltpu.VMEM((DEG_CH,), jnp.float32),
            pltpu.VMEM((16,), jnp.float32),
            pltpu.VMEM_SHARED((N,), jnp.float32),
        ]

    mesh = plsc.VectorSubcoreMesh(core_axis_name="c", subcore_axis_name="s")

    @functools.partial(pl.kernel, out_type=tuple(out_type), mesh=mesh,
                       scratch_types=scratch)
    def sc_agg(h_hbm, src_hbm, dst_hbm, out_sum, *rest):
        if with_deg:
            out_deg0, out_deg1 = rest[0], rest[1]
            rest = rest[2:]
        src_v = rest[0:R]
        dst_v = rest[R:2 * R]
        rows_v = rest[2 * R:3 * R]
        acc = rest[3 * R]
        sem_i = rest[3 * R + 1:4 * R + 1]
        sem_g = rest[4 * R + 1:5 * R + 1]
        sem_s = rest[5 * R + 1:6 * R + 1]
        if with_deg:
            ones_v, dzero, dtail, dacc = rest[6 * R + 1:]
        c = lax.axis_index("c")
        s = lax.axis_index("s")
        wid = c * NS + s
        ebase = wid * EPT

        zero16 = jnp.zeros((16,), jnp.float32)

        def zrow(i, carry):
            for j in range(D // 16):
                rows_v[0][i, pl.ds(j * 16, 16)] = zero16
            return carry

        lax.fori_loop(0, C, zrow, 0)
        for k in range(RCH // C):
            pltpu.sync_copy(rows_v[0], acc.at[pl.ds(s * RCH + k * C, C)])
        rem = RCH % C
        if rem:
            pltpu.sync_copy(rows_v[0].at[pl.ds(0, rem)],
                            acc.at[pl.ds(s * RCH + RCH - rem, rem)])

        @pl.when(s == NS - 1)
        def _():
            pltpu.sync_copy(rows_v[0].at[pl.ds(0, 16)],
                            acc.at[pl.ds(N - 16, 16)])

        if with_deg:
            for j in range(C // 16):
                ones_v[pl.ds(j * 16, 16)] = jnp.ones((16,), jnp.float32)

            def dzrow(i, carry):
                dzero[pl.ds(i * 16, 16)] = zero16
                return carry

            lax.fori_loop(0, DEG_CH // 16, dzrow, 0)
            pltpu.sync_copy(dzero, dacc.at[pl.ds(s * DEG_CH, DEG_CH)])

            @pl.when(s == NS - 1)
            def _():
                pltpu.sync_copy(dzero.at[pl.ds(0, 16)],
                                dacc.at[pl.ds(N - 16, 16)])

        plsc.subcore_barrier()

        def issue_idx(g, j):
            base = ebase + g * C
            pltpu.async_copy(src_hbm.at[pl.ds(base, C)], src_v[j], sem_i[j])
            pltpu.async_copy(dst_hbm.at[pl.ds(base, C)], dst_v[j], sem_i[j])

        def wait_idx(j):
            pltpu.make_async_copy(src_hbm.at[pl.ds(ebase, C)],
                                  src_v[j], sem_i[j]).wait()
            pltpu.make_async_copy(dst_hbm.at[pl.ds(ebase, C)],
                                  dst_v[j], sem_i[j]).wait()

        def issue_gather(j):
            pltpu.async_copy(h_hbm.at[src_v[j]], rows_v[j], sem_g[j])

        def wait_gather(j):
            pltpu.make_async_copy(h_hbm.at[src_v[j]], rows_v[j],
                                  sem_g[j]).wait()

        def scatter(j):
            pltpu.async_copy(rows_v[j], acc.at[dst_v[j]], sem_s[j], add=True)
            if with_deg:
                pltpu.async_copy(ones_v, dacc.at[dst_v[j]], sem_s[j],
                                 add=True)

        def wait_scatter(j):
            pltpu.make_async_copy(rows_v[j], acc.at[dst_v[j]],
                                  sem_s[j]).wait()
            if with_deg:
                pltpu.make_async_copy(ones_v, dacc.at[dst_v[j]],
                                      sem_s[j]).wait()

        for j in range(R):
            issue_idx(j, j)
        for j in range(R):
            wait_idx(j)
            issue_gather(j)
        for j in range(R):
            wait_gather(j)
            scatter(j)

        NBODY = NCHUNK // R - 1

        def body(i, carry):
            gbase = R * (i + 1)
            for j in range(R):
                wait_scatter(j)
                issue_idx(gbase + j, j)
            for j in range(R):
                wait_idx(j)
                issue_gather(j)
            for j in range(R):
                wait_gather(j)
                scatter(j)
            return carry

        lax.fori_loop(0, NBODY, body, 0)

        for t, g in enumerate(range(R * (NCHUNK // R), NCHUNK)):
            j = t % R
            wait_scatter(j)
            issue_idx(g, j)
            wait_idx(j)
            issue_gather(j)
            wait_gather(j)
            scatter(j)
        for j in range(R):
            wait_scatter(j)

        plsc.subcore_barrier()

        pltpu.sync_copy(acc.at[pl.ds(s * RCH, RCH)],
                        out_sum.at[pl.ds(s * RCH, RCH), pl.ds(c * D, D)])

        @pl.when(s == NS - 1)
        def _():
            pltpu.sync_copy(acc.at[pl.ds(N - 16, 16)],
                            out_sum.at[pl.ds(N - 16, 16), pl.ds(c * D, D)])
        if with_deg:
            pltpu.sync_copy(dacc.at[pl.ds(s * DEG_CH, DEG_CH)], dzero)
            for core_id, out_deg in ((0, out_deg0), (1, out_deg1)):
                @pl.when(c == core_id)
                def _(out_deg=out_deg):
                    pltpu.sync_copy(dzero,
                                    out_deg.at[pl.ds(s * DEG_CH, DEG_CH)])

            @pl.when(s == NS - 1)
            def _():
                pltpu.sync_copy(dacc.at[pl.ds(N - 16, 16)],
                                dtail)
                for core_id, out_deg in ((0, out_deg0), (1, out_deg1)):
                    @pl.when(c == core_id)
                    def _(out_deg=out_deg):
                        pltpu.sync_copy(dtail,
                                        out_deg.at[pl.ds(N - 16, 16)])

    return sc_agg


_sc_agg_deg = _make_sc_agg(True, C=80, R=4)
_sc_agg = _make_sc_agg(False, C=80, R=4)

BN = 1000


def _tc_body(sp, h, degt, wlt, wrt, out):
    d = degt[:, 0:1] + degt[:, 1:2] + 1.0
    hh = h[...]
    spb = sp[...]
    agg = (spb[:, :D] + spb[:, D:] + hh) / d
    acc = jnp.dot(hh, wlt[...], preferred_element_type=jnp.float32)
    acc = acc + jnp.dot(agg, wrt[...], preferred_element_type=jnp.float32)
    out[...] = jnp.tanh(acc)


_tc_combine = pl.pallas_call(
    _tc_body,
    grid=(N // BN,),
    in_specs=[
        pl.BlockSpec((BN, NC * D), lambda i: (i, 0)),
        pl.BlockSpec((BN, D), lambda i: (i, 0)),
        pl.BlockSpec((BN, 2), lambda i: (i, 0)),
        pl.BlockSpec((D, D), lambda i: (0, 0)),
        pl.BlockSpec((D, D), lambda i: (0, 0)),
    ],
    out_specs=pl.BlockSpec((BN, D), lambda i: (i, 0)),
    out_shape=jax.ShapeDtypeStruct((N, D), jnp.float32),
)


def kernel(x, edge_index, W1, W2):
    src = edge_index[0]
    dst = edge_index[1]
    sums1, deg0, deg1 = _sc_agg_deg(x, src, dst)
    degt = jnp.stack([deg0, deg1], axis=1)
    h1 = _tc_combine(sums1, x, degt,
                     W1[:, :D].T, W1[:, D:].T)
    (sums2,) = _sc_agg(h1, src, dst)
    h2 = _tc_combine(sums2, h1, degt,
                     W2[:, :D].T, W2[:, D:].T)
    return h2


# speedup vs baseline: 1.0233x; 1.0233x over previous
import functools

import jax
import jax.numpy as jnp
from jax import lax
from jax.experimental import pallas as pl
from jax.experimental.pallas import tpu as pltpu
from jax.experimental.pallas import tpu_sc as plsc

N = 10000
E = 320000
D = 128

NC = 2
NS = 16
NW = NC * NS
EPT = E // NW
RCH = 624
DEG_CH = 624


CH = 128
NCHG = E // CH
NCHT = NCHG // NW
NEXTRA = NCHG - NCHT * NW
RSC = 3
DZ = 208


def _make_sc_agg(with_deg: bool):
    out_type = [jax.ShapeDtypeStruct((N, NC * D), jnp.float32)]
    scratch = (
        [pltpu.VMEM((2, CH), jnp.int32) for _ in range(RSC)]
        + [pltpu.VMEM((CH, D), jnp.float32) for _ in range(RSC)]
        + [pltpu.VMEM_SHARED((N, D), jnp.float32)]
        + [pltpu.SemaphoreType.DMA for _ in range(3 * RSC)]
    )
    if with_deg:
        out_type.append(jax.ShapeDtypeStruct((N,), jnp.float32))
        out_type.append(jax.ShapeDtypeStruct((N,), jnp.float32))
        scratch += [
            pltpu.VMEM((CH,), jnp.float32),
            pltpu.VMEM((DZ,), jnp.float32),
            pltpu.VMEM((16,), jnp.float32),
            pltpu.VMEM_SHARED((N,), jnp.float32),
        ]

    mesh = plsc.VectorSubcoreMesh(core_axis_name="c", subcore_axis_name="s")

    @functools.partial(pl.kernel, out_type=tuple(out_type), mesh=mesh,
                       scratch_types=scratch)
    def sc_agg(h_hbm, e_hbm, out_sum, *rest):
        if with_deg:
            out_deg0, out_deg1 = rest[0], rest[1]
            rest = rest[2:]
        idx_v = rest[0:RSC]
        rows_v = rest[RSC:2 * RSC]
        acc = rest[2 * RSC]
        sem_i = rest[2 * RSC + 1:3 * RSC + 1]
        sem_g = rest[3 * RSC + 1:4 * RSC + 1]
        sem_s = rest[4 * RSC + 1:5 * RSC + 1]
        if with_deg:
            ones_v, dzero, dtail, dacc = rest[5 * RSC + 1:]
        c = lax.axis_index("c")
        s = lax.axis_index("s")
        wid = c * NS + s

        zero16 = jnp.zeros((16,), jnp.float32)

        def zrow(i, carry):
            for j in range(D // 16):
                rows_v[0][i, pl.ds(j * 16, 16)] = zero16
            return carry

        lax.fori_loop(0, CH, zrow, 0)
        for k in range(RCH // CH):
            pltpu.sync_copy(rows_v[0], acc.at[pl.ds(s * RCH + k * CH, CH)])
        rem = RCH % CH
        if rem:
            pltpu.sync_copy(rows_v[0].at[pl.ds(0, rem)],
                            acc.at[pl.ds(s * RCH + RCH - rem, rem)])

        @pl.when(s == NS - 1)
        def _():
            pltpu.sync_copy(rows_v[0].at[pl.ds(0, 16)],
                            acc.at[pl.ds(N - 16, 16)])

        if with_deg:
            for j in range(CH // 16):
                ones_v[pl.ds(j * 16, 16)] = jnp.ones((16,), jnp.float32)

            def dzrow(i, carry):
                dzero[pl.ds(i * 16, 16)] = zero16
                return carry

            lax.fori_loop(0, DZ // 16, dzrow, 0)
            for k in range(DEG_CH // DZ):
                pltpu.sync_copy(dzero,
                                dacc.at[pl.ds(s * DEG_CH + k * DZ, DZ)])

            @pl.when(s == NS - 1)
            def _():
                pltpu.sync_copy(dzero.at[pl.ds(0, 16)],
                                dacc.at[pl.ds(N - 16, 16)])

        plsc.subcore_barrier()

        def issue_idx(gl, j):
            pltpu.async_copy(e_hbm.at[:, pl.ds(gl * CH, CH)], idx_v[j],
                             sem_i[j])

        def wait_idx(j):
            pltpu.make_async_copy(e_hbm.at[:, pl.ds(0, CH)], idx_v[j],
                                  sem_i[j]).wait()

        def issue_gather(j):
            pltpu.async_copy(h_hbm.at[idx_v[j].at[0]], rows_v[j], sem_g[j])

        def wait_gather(j):
            pltpu.make_async_copy(h_hbm.at[idx_v[j].at[0]], rows_v[j],
                                  sem_g[j]).wait()

        def scatter(j):
            pltpu.async_copy(rows_v[j], acc.at[idx_v[j].at[1]], sem_s[j],
                             add=True)
            if with_deg:
                pltpu.async_copy(ones_v, dacc.at[idx_v[j].at[1]], sem_s[j],
                                 add=True)

        def wait_scatter(j):
            pltpu.make_async_copy(rows_v[j], acc.at[idx_v[j].at[1]],
                                  sem_s[j]).wait()
            if with_deg:
                pltpu.make_async_copy(ones_v, dacc.at[idx_v[j].at[1]],
                                      sem_s[j]).wait()

        gl0 = wid * NCHT

        for j in range(RSC):
            issue_idx(gl0 + j, j)
        for j in range(RSC):
            wait_idx(j)
            issue_gather(j)
        for j in range(RSC):
            wait_gather(j)
            scatter(j)

        NBODY = NCHT // RSC - 1

        def body(i, carry):
            gbase = gl0 + RSC * (i + 1)
            for j in range(RSC):
                wait_scatter(j)
                issue_idx(gbase + j, j)
            for j in range(RSC):
                wait_idx(j)
                issue_gather(j)
            for j in range(RSC):
                wait_gather(j)
                scatter(j)
            return carry

        lax.fori_loop(0, NBODY, body, 0)

        for t, g in enumerate(range(RSC * (NCHT // RSC), NCHT)):
            j = t % RSC
            wait_scatter(j)
            issue_idx(gl0 + g, j)
            wait_idx(j)
            issue_gather(j)
            wait_gather(j)
            scatter(j)
        for j in range(RSC):
            wait_scatter(j)

        @pl.when(wid < NEXTRA)
        def _():
            issue_idx(NCHT * NW + wid, 0)
            wait_idx(0)
            issue_gather(0)
            wait_gather(0)
            scatter(0)
            wait_scatter(0)

        plsc.subcore_barrier()

        pltpu.sync_copy(acc.at[pl.ds(s * RCH, RCH)],
                        out_sum.at[pl.ds(s * RCH, RCH), pl.ds(c * D, D)])

        @pl.when(s == NS - 1)
        def _():
            pltpu.sync_copy(acc.at[pl.ds(N - 16, 16)],
                            out_sum.at[pl.ds(N - 16, 16), pl.ds(c * D, D)])
        if with_deg:
            for core_id, out_deg in ((0, out_deg0), (1, out_deg1)):
                @pl.when(c == core_id)
                def _(out_deg=out_deg):
                    for k in range(DEG_CH // DZ):
                        pltpu.sync_copy(
                            dacc.at[pl.ds(s * DEG_CH + k * DZ, DZ)], dzero)
                        pltpu.sync_copy(
                            dzero, out_deg.at[pl.ds(s * DEG_CH + k * DZ, DZ)])

            @pl.when(s == NS - 1)
            def _():
                pltpu.sync_copy(dacc.at[pl.ds(N - 16, 16)], dtail)
                for core_id, out_deg in ((0, out_deg0), (1, out_deg1)):
                    @pl.when(c == core_id)
                    def _(out_deg=out_deg):
                        pltpu.sync_copy(dtail,
                                        out_deg.at[pl.ds(N - 16, 16)])

    return sc_agg


_sc_agg_deg = _make_sc_agg(True)
_sc_agg = _make_sc_agg(False)

BN = 1000


def _tc_body(sp, h, degt, wlt, wrt, out):
    d = degt[:, 0:1] + degt[:, 1:2] + 1.0
    hh = h[...]
    spb = sp[...]
    agg = (spb[:, :D] + spb[:, D:] + hh) / d
    acc = jnp.dot(hh, wlt[...], preferred_element_type=jnp.float32)
    acc = acc + jnp.dot(agg, wrt[...], preferred_element_type=jnp.float32)
    out[...] = jnp.tanh(acc)


_tc_combine = pl.pallas_call(
    _tc_body,
    grid=(N // BN,),
    in_specs=[
        pl.BlockSpec((BN, NC * D), lambda i: (i, 0)),
        pl.BlockSpec((BN, D), lambda i: (i, 0)),
        pl.BlockSpec((BN, 2), lambda i: (i, 0)),
        pl.BlockSpec((D, D), lambda i: (0, 0)),
        pl.BlockSpec((D, D), lambda i: (0, 0)),
    ],
    out_specs=pl.BlockSpec((BN, D), lambda i: (i, 0)),
    out_shape=jax.ShapeDtypeStruct((N, D), jnp.float32),
)


def kernel(x, edge_index, W1, W2):
    sums1, deg0, deg1 = _sc_agg_deg(x, edge_index)
    degt = jnp.stack([deg0, deg1], axis=1)
    h1 = _tc_combine(sums1, x, degt,
                     W1[:, :D].T, W1[:, D:].T)
    (sums2,) = _sc_agg(h1, edge_index)
    h2 = _tc_combine(sums2, h1, degt,
                     W2[:, :D].T, W2[:, D:].T)
    return h2


# speedup vs baseline: 1.0392x; 1.0155x over previous
import functools

import jax
import jax.numpy as jnp
from jax import lax
from jax.experimental import pallas as pl
from jax.experimental.pallas import tpu as pltpu
from jax.experimental.pallas import tpu_sc as plsc

N = 10000
E = 320000
D = 128

NC = 2
NS = 16
NW = NC * NS
EPT = E // NW
RCH = 624
DEG_CH = 624


CH = 128
NCHG = E // CH
NCHT = NCHG // NW
NEXTRA = NCHG - NCHT * NW
RSC = 3
DZ = 208


def _make_sc_agg(with_deg: bool):
    out_type = [jax.ShapeDtypeStruct((N, NC * D), jnp.float32)]
    scratch = (
        [pltpu.VMEM((2, CH), jnp.int32) for _ in range(RSC)]
        + [pltpu.VMEM((CH, D), jnp.float32) for _ in range(RSC)]
        + [pltpu.VMEM_SHARED((N, D), jnp.float32)]
        + [pltpu.SemaphoreType.DMA for _ in range(3 * RSC)]
    )
    if with_deg:
        out_type.append(jax.ShapeDtypeStruct((N,), jnp.float32))
        out_type.append(jax.ShapeDtypeStruct((N,), jnp.float32))
        scratch += [
            pltpu.VMEM((CH,), jnp.float32),
            pltpu.VMEM((DZ,), jnp.float32),
            pltpu.VMEM((16,), jnp.float32),
            pltpu.VMEM_SHARED((N,), jnp.float32),
        ]

    mesh = plsc.VectorSubcoreMesh(core_axis_name="c", subcore_axis_name="s")

    @functools.partial(pl.kernel, out_type=tuple(out_type), mesh=mesh,
                       scratch_types=scratch)
    def sc_agg(h_hbm, e_hbm, out_sum, *rest):
        if with_deg:
            out_deg0, out_deg1 = rest[0], rest[1]
            rest = rest[2:]
        idx_v = rest[0:RSC]
        rows_v = rest[RSC:2 * RSC]
        acc = rest[2 * RSC]
        sem_i = rest[2 * RSC + 1:3 * RSC + 1]
        sem_g = rest[3 * RSC + 1:4 * RSC + 1]
        sem_s = rest[4 * RSC + 1:5 * RSC + 1]
        if with_deg:
            ones_v, dzero, dtail, dacc = rest[5 * RSC + 1:]
        c = lax.axis_index("c")
        s = lax.axis_index("s")
        wid = c * NS + s

        zero16 = jnp.zeros((16,), jnp.float32)

        def zrow(i, carry):
            for j in range(D // 16):
                rows_v[0][i, pl.ds(j * 16, 16)] = zero16
            return carry

        lax.fori_loop(0, CH, zrow, 0)
        for k in range(RCH // CH):
            pltpu.sync_copy(rows_v[0], acc.at[pl.ds(s * RCH + k * CH, CH)])
        rem = RCH % CH
        if rem:
            pltpu.sync_copy(rows_v[0].at[pl.ds(0, rem)],
                            acc.at[pl.ds(s * RCH + RCH - rem, rem)])

        @pl.when(s == NS - 1)
        def _():
            pltpu.sync_copy(rows_v[0].at[pl.ds(0, 16)],
                            acc.at[pl.ds(N - 16, 16)])

        if with_deg:
            for j in range(CH // 16):
                ones_v[pl.ds(j * 16, 16)] = jnp.ones((16,), jnp.float32)

            def dzrow(i, carry):
                dzero[pl.ds(i * 16, 16)] = zero16
                return carry

            lax.fori_loop(0, DZ // 16, dzrow, 0)
            for k in range(DEG_CH // DZ):
                pltpu.sync_copy(dzero,
                                dacc.at[pl.ds(s * DEG_CH + k * DZ, DZ)])

            @pl.when(s == NS - 1)
            def _():
                pltpu.sync_copy(dzero.at[pl.ds(0, 16)],
                                dacc.at[pl.ds(N - 16, 16)])

        plsc.subcore_barrier()

        def issue_idx(gl, j):
            pltpu.async_copy(e_hbm.at[:, pl.ds(gl * CH, CH)], idx_v[j],
                             sem_i[j])

        def wait_idx(j):
            pltpu.make_async_copy(e_hbm.at[:, pl.ds(0, CH)], idx_v[j],
                                  sem_i[j]).wait()

        def issue_gather(j):
            pltpu.async_copy(h_hbm.at[idx_v[j].at[0]], rows_v[j], sem_g[j])

        def wait_gather(j):
            pltpu.make_async_copy(h_hbm.at[idx_v[j].at[0]], rows_v[j],
                                  sem_g[j]).wait()

        def scatter(j):
            pltpu.async_copy(rows_v[j], acc.at[idx_v[j].at[1]], sem_s[j],
                             add=True)
            if with_deg:
                pltpu.async_copy(ones_v, dacc.at[idx_v[j].at[1]], sem_s[j],
                                 add=True)

        def wait_scatter(j):
            pltpu.make_async_copy(rows_v[j], acc.at[idx_v[j].at[1]],
                                  sem_s[j]).wait()
            if with_deg:
                pltpu.make_async_copy(ones_v, dacc.at[idx_v[j].at[1]],
                                      sem_s[j]).wait()

        gl0 = wid * NCHT

        for j in range(RSC):
            issue_idx(gl0 + j, j)
        for j in range(RSC):
            wait_idx(j)
            issue_gather(j)
        for j in range(RSC):
            wait_gather(j)
            scatter(j)

        NBODY = NCHT // RSC - 1

        def body(i, carry):
            gbase = gl0 + RSC * (i + 1)
            for j in range(RSC):
                wait_scatter(j)
                issue_idx(gbase + j, j)
            for j in range(RSC):
                wait_idx(j)
                issue_gather(j)
            for j in range(RSC):
                wait_gather(j)
                scatter(j)
            return carry

        lax.fori_loop(0, NBODY, body, 0)

        for t, g in enumerate(range(RSC * (NCHT // RSC), NCHT)):
            j = t % RSC
            wait_scatter(j)
            issue_idx(gl0 + g, j)
            wait_idx(j)
            issue_gather(j)
            wait_gather(j)
            scatter(j)
        for j in range(RSC):
            wait_scatter(j)

        @pl.when(wid < NEXTRA)
        def _():
            issue_idx(NCHT * NW + wid, 0)
            wait_idx(0)
            issue_gather(0)
            wait_gather(0)
            scatter(0)
            wait_scatter(0)

        plsc.subcore_barrier()

        pltpu.sync_copy(acc.at[pl.ds(s * RCH, RCH)],
                        out_sum.at[pl.ds(s * RCH, RCH), pl.ds(c * D, D)])

        @pl.when(s == NS - 1)
        def _():
            pltpu.sync_copy(acc.at[pl.ds(N - 16, 16)],
                            out_sum.at[pl.ds(N - 16, 16), pl.ds(c * D, D)])
        if with_deg:
            for core_id, out_deg in ((0, out_deg0), (1, out_deg1)):
                @pl.when(c == core_id)
                def _(out_deg=out_deg):
                    for k in range(DEG_CH // DZ):
                        pltpu.sync_copy(
                            dacc.at[pl.ds(s * DEG_CH + k * DZ, DZ)], dzero)
                        pltpu.sync_copy(
                            dzero, out_deg.at[pl.ds(s * DEG_CH + k * DZ, DZ)])

            @pl.when(s == NS - 1)
            def _():
                pltpu.sync_copy(dacc.at[pl.ds(N - 16, 16)], dtail)
                for core_id, out_deg in ((0, out_deg0), (1, out_deg1)):
                    @pl.when(c == core_id)
                    def _(out_deg=out_deg):
                        pltpu.sync_copy(dtail,
                                        out_deg.at[pl.ds(N - 16, 16)])

    return sc_agg


_sc_agg_deg = _make_sc_agg(True)
_sc_agg = _make_sc_agg(False)

BN = 2000


def _tc_body(sp, h, degt, wlt, wrt, out):
    d = degt[:, 0:1] + degt[:, 1:2] + 1.0
    hh = h[...]
    spb = sp[...]
    agg = (spb[:, :D] + spb[:, D:] + hh) / d
    acc = jnp.dot(hh, wlt[...], preferred_element_type=jnp.float32)
    acc = acc + jnp.dot(agg, wrt[...], preferred_element_type=jnp.float32)
    out[...] = jnp.tanh(acc)


_tc_combine = pl.pallas_call(
    _tc_body,
    grid=(N // BN,),
    in_specs=[
        pl.BlockSpec((BN, NC * D), lambda i: (i, 0)),
        pl.BlockSpec((BN, D), lambda i: (i, 0)),
        pl.BlockSpec((BN, 2), lambda i: (i, 0)),
        pl.BlockSpec((D, D), lambda i: (0, 0)),
        pl.BlockSpec((D, D), lambda i: (0, 0)),
    ],
    out_specs=pl.BlockSpec((BN, D), lambda i: (i, 0)),
    out_shape=jax.ShapeDtypeStruct((N, D), jnp.float32),
)


def kernel(x, edge_index, W1, W2):
    sums1, deg0, deg1 = _sc_agg_deg(x, edge_index)
    degt = jnp.stack([deg0, deg1], axis=1)
    h1 = _tc_combine(sums1, x, degt,
                     W1[:, :D].T, W1[:, D:].T)
    (sums2,) = _sc_agg(h1, edge_index)
    h2 = _tc_combine(sums2, h1, degt,
                     W2[:, :D].T, W2[:, D:].T)
    return h2
